# Initial kernel scaffold; baseline (speedup 1.0000x reference)
import jax
import jax.numpy as jnp
from jax.experimental import pallas as pl


def kernel(x, c, w_router, w_ada, b_ada, w_qkv, b_qkv, w_proj, b_proj, w_fc1, b_fc1, w_fc2, b_fc2, activate_mod_router):
    raise NotImplementedError("write your pallas kernel here")


# speedup vs baseline: 3.6637x; 3.6637x over previous
import functools

import jax
import jax.numpy as jnp
from jax import lax
from jax.experimental import pallas as pl
from jax.experimental.pallas import tpu as pltpu
from jax.experimental.pallas import tpu_sc as plsc

B, N, C, H = 4, 8192, 768, 12
HD = C // H
K = max(1, min(N, int(0.1 * N)))
KP = ((K + 63) // 64) * 64
TOT = B * KP
NC, NS = 2, 16
NW = NC * NS
PER_W = TOT // NW
NB = 512
_VMEM_LIM = 128 * 1024 * 1024


def _ln(t):
    mu = jnp.mean(t, axis=-1, keepdims=True)
    var = jnp.mean((t - mu) ** 2, axis=-1, keepdims=True)
    return (t - mu) / jnp.sqrt(var + 1e-6)


def _gelu_tanh(t):
    return 0.5 * t * (1.0 + jnp.tanh(jnp.sqrt(2.0 / jnp.pi) * (t + 0.044715 * t ** 3)))


def _k1_body(x_ref, wr_ref, cp_ref, tw_ref):
    cp_ref[...] = x_ref[...]
    tw_ref[0] = jnp.dot(x_ref[0], wr_ref[...], preferred_element_type=jnp.float32)


def _router_copy(x, w_router):
    return pl.pallas_call(
        _k1_body,
        grid=(B, N // NB),
        in_specs=[
            pl.BlockSpec((1, NB, C), lambda b, n: (b, n, 0)),
            pl.BlockSpec((C, 1), lambda b, n: (0, 0)),
        ],
        out_specs=[
            pl.BlockSpec((1, NB, C), lambda b, n: (b, n, 0)),
            pl.BlockSpec((1, NB, 1), lambda b, n: (b, n, 0)),
        ],
        out_shape=[
            jax.ShapeDtypeStruct((B, N, C), jnp.float32),
            jax.ShapeDtypeStruct((B, N, 1), jnp.float32),
        ],
    )(x, w_router)


def _d0_body(c_ref, wa_ref, ba_ref, out_ref):
    cc = c_ref[...]
    silu = cc * jax.nn.sigmoid(cc)
    out_ref[...] = jnp.dot(silu, wa_ref[...], preferred_element_type=jnp.float32) + ba_ref[...]


def _ada(c, w_ada, b_ada):
    return pl.pallas_call(
        _d0_body,
        out_shape=jax.ShapeDtypeStruct((B, 6 * C), jnp.float32),
    )(c, w_ada, b_ada.reshape(1, 6 * C))


def _d1_body(tx_ref, sh_ref, sc_ref, wq_ref, bq_ref, out_ref):
    tx = tx_ref[0]
    h = _ln(tx) * (1.0 + sc_ref[0]) + sh_ref[0]
    out_ref[0] = jnp.dot(h, wq_ref[...], preferred_element_type=jnp.float32) + bq_ref[...]


def _qkv(topk_x, shift_msa, scale_msa, w_qkv, b_qkv):
    return pl.pallas_call(
        _d1_body,
        grid=(B,),
        in_specs=[
            pl.BlockSpec((1, KP, C), lambda b: (b, 0, 0)),
            pl.BlockSpec((1, 1, C), lambda b: (b, 0, 0)),
            pl.BlockSpec((1, 1, C), lambda b: (b, 0, 0)),
            pl.BlockSpec((C, 3 * C), lambda b: (0, 0)),
            pl.BlockSpec((1, 3 * C), lambda b: (0, 0)),
        ],
        out_specs=pl.BlockSpec((1, KP, 3 * C), lambda b: (b, 0, 0)),
        out_shape=jax.ShapeDtypeStruct((B, KP, 3 * C), jnp.float32),
        compiler_params=pltpu.CompilerParams(vmem_limit_bytes=_VMEM_LIM),
    )(topk_x, shift_msa, scale_msa, w_qkv, b_qkv.reshape(1, 3 * C))


def _d2_body(q_ref, k_ref, v_ref, o_ref):
    scale = HD ** -0.5
    outs = []
    for s in (0, HD):
        q = q_ref[0, :, s:s + HD] * scale
        k = k_ref[0, :, s:s + HD]
        v = v_ref[0, :, s:s + HD]
        st = lax.dot_general(q, k, (((1,), (1,)), ((), ())),
                             preferred_element_type=jnp.float32)
        col = lax.broadcasted_iota(jnp.int32, (KP, KP), 1)
        st = jnp.where(col < K, st, -1e30)
        m = jnp.max(st, axis=-1, keepdims=True)
        p = jnp.exp(st - m)
        p = p / jnp.sum(p, axis=-1, keepdims=True)
        outs.append(jnp.dot(p, v, preferred_element_type=jnp.float32))
    o_ref[0] = jnp.concatenate(outs, axis=1)


def _attention(qkv):
    blk = 2 * HD
    return pl.pallas_call(
        _d2_body,
        grid=(B, H // 2),
        in_specs=[
            pl.BlockSpec((1, KP, blk), lambda b, h: (b, 0, h)),
            pl.BlockSpec((1, KP, blk), lambda b, h: (b, 0, (C // blk) + h)),
            pl.BlockSpec((1, KP, blk), lambda b, h: (b, 0, 2 * (C // blk) + h)),
        ],
        out_specs=pl.BlockSpec((1, KP, blk), lambda b, h: (b, 0, h)),
        out_shape=jax.ShapeDtypeStruct((B, KP, C), jnp.float32),
        compiler_params=pltpu.CompilerParams(vmem_limit_bytes=_VMEM_LIM),
    )(qkv, qkv, qkv)


def _d3_body(tx_ref, o_ref, gm_ref, shm_ref, scm_ref, gl_ref, w_ref,
             wp_ref, bp_ref, w1_ref, b1_ref, w2_ref, b2_ref, out_ref):
    tx = tx_ref[0]
    t2 = tx + gm_ref[0] * (
        jnp.dot(o_ref[0], wp_ref[...], preferred_element_type=jnp.float32) + bp_ref[...])
    h2 = _ln(t2) * (1.0 + scm_ref[0]) + shm_ref[0]
    hid = _gelu_tanh(jnp.dot(h2, w1_ref[...], preferred_element_type=jnp.float32) + b1_ref[...])
    m = jnp.dot(hid, w2_ref[...], preferred_element_type=jnp.float32) + b2_ref[...]
    out_ref[0] = t2 + w_ref[0] * (gl_ref[0] * m)


def _mlp(topk_x, o, gate_msa, shift_mlp, scale_mlp, gate_mlp, w_vals,
         w_proj, b_proj, w_fc1, b_fc1, w_fc2, b_fc2):
    vec = pl.BlockSpec((1, 1, C), lambda b: (b, 0, 0))
    return pl.pallas_call(
        _d3_body,
        grid=(B,),
        in_specs=[
            pl.BlockSpec((1, KP, C), lambda b: (b, 0, 0)),
            pl.BlockSpec((1, KP, C), lambda b: (b, 0, 0)),
            vec, vec, vec, vec,
            pl.BlockSpec((1, KP, 1), lambda b: (b, 0, 0)),
            pl.BlockSpec((C, C), lambda b: (0, 0)),
            pl.BlockSpec((1, C), lambda b: (0, 0)),
            pl.BlockSpec((C, 4 * C), lambda b: (0, 0)),
            pl.BlockSpec((1, 4 * C), lambda b: (0, 0)),
            pl.BlockSpec((4 * C, C), lambda b: (0, 0)),
            pl.BlockSpec((1, C), lambda b: (0, 0)),
        ],
        out_specs=pl.BlockSpec((1, KP, C), lambda b: (b, 0, 0)),
        out_shape=jax.ShapeDtypeStruct((B, KP, C), jnp.float32),
        compiler_params=pltpu.CompilerParams(vmem_limit_bytes=_VMEM_LIM),
    )(topk_x, o, gate_msa, shift_mlp, scale_mlp, gate_mlp, w_vals,
      w_proj, b_proj.reshape(1, C), w_fc1, b_fc1.reshape(1, 4 * C),
      w_fc2, b_fc2.reshape(1, C))


@functools.lru_cache(maxsize=1)
def _sc_kernels():
    mesh = plsc.VectorSubcoreMesh(
        core_axis_name="c", subcore_axis_name="s",
        num_cores=NC, num_subcores=NS)
    scratch = [
        pltpu.VMEM((PER_W,), jnp.int32),
        pltpu.VMEM((PER_W, C), jnp.float32),
        pltpu.SemaphoreType.DMA,
    ]

    @functools.partial(
        pl.kernel,
        out_type=jax.ShapeDtypeStruct((TOT, C), jnp.float32),
        mesh=mesh,
        scratch_types=scratch,
    )
    def gather(table_hbm, idx_hbm, out_hbm, idx_v, rows_v, sem):
        wid = lax.axis_index("s") * NC + lax.axis_index("c")
        base = wid * PER_W
        pltpu.sync_copy(idx_hbm.at[pl.ds(base, PER_W)], idx_v)
        pltpu.async_copy(table_hbm.at[idx_v], rows_v, sem).wait()
        pltpu.sync_copy(rows_v, out_hbm.at[pl.ds(base, PER_W)])

    @functools.partial(pl.kernel, out_type=(), mesh=mesh, scratch_types=scratch)
    def scatter(rows_hbm, idx_hbm, dst_ref, idx_v, rows_v, sem):
        wid = lax.axis_index("s") * NC + lax.axis_index("c")
        base = wid * PER_W
        pltpu.sync_copy(idx_hbm.at[pl.ds(base, PER_W)], idx_v)
        pltpu.sync_copy(rows_hbm.at[pl.ds(base, PER_W)], rows_v)
        pltpu.async_copy(rows_v, dst_ref.at[idx_v], sem).wait()

    return gather, scatter


def _sc_gather(table, gidx):
    return _sc_kernels()[0](table, gidx)


def _sc_scatter(rows, gidx, dst_ref):
    _sc_kernels()[1](rows, gidx, dst_ref)


def kernel(x, c, w_router, w_ada, b_ada, w_qkv, b_qkv, w_proj, b_proj,
           w_fc1, b_fc1, w_fc2, b_fc2, activate_mod_router):
    cp, tw3 = _router_copy(x, w_router)
    tw = tw3[..., 0]
    vals, idx = lax.top_k(tw, K)
    w = jax.nn.sigmoid(vals)
    pad = KP - K
    idx_p = jnp.concatenate([idx, jnp.broadcast_to(idx[:, -1:], (B, pad))], axis=1)
    w_p = jnp.concatenate([w, jnp.broadcast_to(w[:, -1:], (B, pad))], axis=1)
    gidx = (idx_p + (jnp.arange(B, dtype=jnp.int32) * N)[:, None]).reshape(TOT)

    rows = _sc_gather(x.reshape(B * N, C), gidx)
    topk_x = rows.reshape(B, KP, C)

    ada6 = _ada(c, w_ada, b_ada).reshape(B, 6, C)
    shift_msa = ada6[:, 0].reshape(B, 1, C)
    scale_msa = ada6[:, 1].reshape(B, 1, C)
    gate_msa = ada6[:, 2].reshape(B, 1, C)
    shift_mlp = ada6[:, 3].reshape(B, 1, C)
    scale_mlp = ada6[:, 4].reshape(B, 1, C)
    gate_mlp = ada6[:, 5].reshape(B, 1, C)

    qkv = _qkv(topk_x, shift_msa, scale_msa, w_qkv, b_qkv)
    o = _attention(qkv)
    out_rows = _mlp(topk_x, o, gate_msa, shift_mlp, scale_mlp, gate_mlp,
                    w_p.reshape(B, KP, 1), w_proj, b_proj, w_fc1, b_fc1,
                    w_fc2, b_fc2)

    buf = jax.new_ref(cp.reshape(B * N, C))
    _sc_scatter(out_rows.reshape(TOT, C), gidx, buf)
    return buf[...].reshape(B, N, C)


# speedup vs baseline: 3.8960x; 1.0634x over previous
import functools

import jax
import jax.numpy as jnp
from jax import lax
from jax.experimental import pallas as pl
from jax.experimental.pallas import tpu as pltpu
from jax.experimental.pallas import tpu_sc as plsc

B, N, C, H = 4, 8192, 768, 12
HD = C // H
K = max(1, min(N, int(0.1 * N)))
KP = ((K + 63) // 64) * 64
TOT = B * KP
NC, NS = 2, 16
NW = NC * NS
PER_W = TOT // NW
ROWS_W = (B * N) // NW
CCH = 64
NCH = ROWS_W // CCH
NB = 1024
_VMEM_LIM = 128 * 1024 * 1024


def _ln(t):
    mu = jnp.mean(t, axis=-1, keepdims=True)
    var = jnp.mean((t - mu) ** 2, axis=-1, keepdims=True)
    return (t - mu) / jnp.sqrt(var + 1e-6)


def _gelu_tanh(t):
    return 0.5 * t * (1.0 + jnp.tanh(jnp.sqrt(2.0 / jnp.pi) * (t + 0.044715 * t ** 3)))


def _k1_body(x_ref, wr_ref, tw_ref):
    tw_ref[0] = jnp.dot(x_ref[0], wr_ref[...], preferred_element_type=jnp.float32)


def _router(x, w_router):
    return pl.pallas_call(
        _k1_body,
        grid=(B, N // NB),
        in_specs=[
            pl.BlockSpec((1, NB, C), lambda b, n: (b, n, 0)),
            pl.BlockSpec((C, 1), lambda b, n: (0, 0)),
        ],
        out_specs=pl.BlockSpec((1, NB, 1), lambda b, n: (b, n, 0)),
        out_shape=jax.ShapeDtypeStruct((B, N, 1), jnp.float32),
    )(x, w_router)


def _d1_body(tx_ref, c_ref, wa_ref, ba_ref, wq_ref, bq_ref, qkv_ref, ada_ref):
    cb = c_ref[0]
    silu = cb * jax.nn.sigmoid(cb)
    ada = jnp.dot(silu, wa_ref[...], preferred_element_type=jnp.float32) + ba_ref[...]
    shift = ada[:, 0:C]
    scale = ada[:, C:2 * C]
    tx = tx_ref[0]
    h = _ln(tx) * (1.0 + scale) + shift
    qkv_ref[0] = jnp.dot(h, wq_ref[...], preferred_element_type=jnp.float32) + bq_ref[...]
    ada_ref[0] = ada


def _qkv(topk_x, c, w_ada, b_ada, w_qkv, b_qkv):
    return pl.pallas_call(
        _d1_body,
        grid=(B,),
        in_specs=[
            pl.BlockSpec((1, KP, C), lambda b: (b, 0, 0)),
            pl.BlockSpec((1, 1, C), lambda b: (b, 0, 0)),
            pl.BlockSpec((C, 6 * C), lambda b: (0, 0)),
            pl.BlockSpec((1, 6 * C), lambda b: (0, 0)),
            pl.BlockSpec((C, 3 * C), lambda b: (0, 0)),
            pl.BlockSpec((1, 3 * C), lambda b: (0, 0)),
        ],
        out_specs=[
            pl.BlockSpec((1, KP, 3 * C), lambda b: (b, 0, 0)),
            pl.BlockSpec((1, 1, 6 * C), lambda b: (b, 0, 0)),
        ],
        out_shape=[
            jax.ShapeDtypeStruct((B, KP, 3 * C), jnp.float32),
            jax.ShapeDtypeStruct((B, 1, 6 * C), jnp.float32),
        ],
        compiler_params=pltpu.CompilerParams(vmem_limit_bytes=_VMEM_LIM),
    )(topk_x, c, w_ada, b_ada.reshape(1, 6 * C), w_qkv, b_qkv.reshape(1, 3 * C))


def _d2_body(q_ref, k_ref, v_ref, o_ref):
    scale = HD ** -0.5
    outs = []
    for s in (0, HD):
        q = q_ref[0, :, s:s + HD] * scale
        k = k_ref[0, :, s:s + HD]
        v = v_ref[0, :, s:s + HD]
        st = lax.dot_general(q, k, (((1,), (1,)), ((), ())),
                             preferred_element_type=jnp.float32)
        col = lax.broadcasted_iota(jnp.int32, (KP, KP), 1)
        st = jnp.where(col < K, st, -1e30)
        m = jnp.max(st, axis=-1, keepdims=True)
        p = jnp.exp(st - m)
        p = p / jnp.sum(p, axis=-1, keepdims=True)
        outs.append(jnp.dot(p, v, preferred_element_type=jnp.float32))
    o_ref[0] = jnp.concatenate(outs, axis=1)


def _attention(qkv):
    blk = 2 * HD
    return pl.pallas_call(
        _d2_body,
        grid=(B, H // 2),
        in_specs=[
            pl.BlockSpec((1, KP, blk), lambda b, h: (b, 0, h)),
            pl.BlockSpec((1, KP, blk), lambda b, h: (b, 0, (C // blk) + h)),
            pl.BlockSpec((1, KP, blk), lambda b, h: (b, 0, 2 * (C // blk) + h)),
        ],
        out_specs=pl.BlockSpec((1, KP, blk), lambda b, h: (b, 0, h)),
        out_shape=jax.ShapeDtypeStruct((B, KP, C), jnp.float32),
        compiler_params=pltpu.CompilerParams(vmem_limit_bytes=_VMEM_LIM),
    )(qkv, qkv, qkv)


def _d3_body(tx_ref, o_ref, ada_ref, v_ref,
             wp_ref, bp_ref, w1_ref, b1_ref, w2_ref, b2_ref, out_ref):
    ada = ada_ref[0]
    gm = ada[:, 2 * C:3 * C]
    shm = ada[:, 3 * C:4 * C]
    scm = ada[:, 4 * C:5 * C]
    gl = ada[:, 5 * C:6 * C]
    tx = tx_ref[0]
    t2 = tx + gm * (
        jnp.dot(o_ref[0], wp_ref[...], preferred_element_type=jnp.float32) + bp_ref[...])
    h2 = _ln(t2) * (1.0 + scm) + shm
    hid = _gelu_tanh(jnp.dot(h2, w1_ref[...], preferred_element_type=jnp.float32) + b1_ref[...])
    m = jnp.dot(hid, w2_ref[...], preferred_element_type=jnp.float32) + b2_ref[...]
    w = jax.nn.sigmoid(v_ref[0])
    out_ref[0] = t2 + w * (gl * m)


def _mlp(topk_x, o, ada, vals, w_proj, b_proj, w_fc1, b_fc1, w_fc2, b_fc2):
    return pl.pallas_call(
        _d3_body,
        grid=(B,),
        in_specs=[
            pl.BlockSpec((1, KP, C), lambda b: (b, 0, 0)),
            pl.BlockSpec((1, KP, C), lambda b: (b, 0, 0)),
            pl.BlockSpec((1, 1, 6 * C), lambda b: (b, 0, 0)),
            pl.BlockSpec((1, KP, 1), lambda b: (b, 0, 0)),
            pl.BlockSpec((C, C), lambda b: (0, 0)),
            pl.BlockSpec((1, C), lambda b: (0, 0)),
            pl.BlockSpec((C, 4 * C), lambda b: (0, 0)),
            pl.BlockSpec((1, 4 * C), lambda b: (0, 0)),
            pl.BlockSpec((4 * C, C), lambda b: (0, 0)),
            pl.BlockSpec((1, C), lambda b: (0, 0)),
        ],
        out_specs=pl.BlockSpec((1, KP, C), lambda b: (b, 0, 0)),
        out_shape=jax.ShapeDtypeStruct((B, KP, C), jnp.float32),
        compiler_params=pltpu.CompilerParams(vmem_limit_bytes=_VMEM_LIM),
    )(topk_x, o, ada, vals, w_proj, b_proj.reshape(1, C), w_fc1,
      b_fc1.reshape(1, 4 * C), w_fc2, b_fc2.reshape(1, C))


@functools.lru_cache(maxsize=1)
def _sc_kernels():
    mesh = plsc.VectorSubcoreMesh(
        core_axis_name="c", subcore_axis_name="s",
        num_cores=NC, num_subcores=NS)
    idx_scratch = [
        pltpu.VMEM((PER_W,), jnp.int32),
        pltpu.VMEM((PER_W, C), jnp.float32),
        pltpu.SemaphoreType.DMA,
    ]

    @functools.partial(
        pl.kernel,
        out_type=jax.ShapeDtypeStruct((TOT, C), jnp.float32),
        mesh=mesh,
        scratch_types=idx_scratch,
    )
    def gather(table_hbm, idx_hbm, out_hbm, idx_v, rows_v, sem):
        wid = lax.axis_index("s") * NC + lax.axis_index("c")
        base = wid * PER_W
        pltpu.sync_copy(idx_hbm.at[pl.ds(base, PER_W)], idx_v)
        pltpu.async_copy(table_hbm.at[idx_v], rows_v, sem).wait()
        pltpu.sync_copy(rows_v, out_hbm.at[pl.ds(base, PER_W)])

    @functools.partial(pl.kernel, out_type=(), mesh=mesh, scratch_types=idx_scratch)
    def scatter(rows_hbm, idx_hbm, dst_ref, idx_v, rows_v, sem):
        wid = lax.axis_index("s") * NC + lax.axis_index("c")
        base = wid * PER_W
        pltpu.sync_copy(idx_hbm.at[pl.ds(base, PER_W)], idx_v)
        pltpu.sync_copy(rows_hbm.at[pl.ds(base, PER_W)], rows_v)
        pltpu.async_copy(rows_v, dst_ref.at[idx_v], sem).wait()

    @functools.partial(
        pl.kernel,
        out_type=jax.ShapeDtypeStruct((B * N, C), jnp.float32),
        mesh=mesh,
        scratch_types=[
            pltpu.VMEM((CCH, C), jnp.float32),
            pltpu.VMEM((CCH, C), jnp.float32),
            pltpu.SemaphoreType.DMA,
            pltpu.SemaphoreType.DMA,
            pltpu.SemaphoreType.DMA,
            pltpu.SemaphoreType.DMA,
        ],
    )
    def copy(src_hbm, dst_hbm, b0, b1, si0, si1, so0, so1):
        wid = lax.axis_index("s") * NC + lax.axis_index("c")
        base = wid * ROWS_W
        bufs = (b0, b1)
        sin = (si0, si1)
        sout = (so0, so1)
        ins = {}
        outs = {}

        def start_in(i):
            ins[i] = pltpu.async_copy(
                src_hbm.at[pl.ds(base + i * CCH, CCH)], bufs[i % 2], sin[i % 2])

        start_in(0)
        for i in range(NCH):
            if i + 1 < NCH:
                if i >= 1:
                    outs[i - 1].wait()
                start_in(i + 1)
            ins[i].wait()
            outs[i] = pltpu.async_copy(
                bufs[i % 2], dst_hbm.at[pl.ds(base + i * CCH, CCH)], sout[i % 2])
        outs[NCH - 2].wait()
        outs[NCH - 1].wait()

    return gather, scatter, copy


def _sc_gather(table, gidx):
    return _sc_kernels()[0](table, gidx)


def _sc_scatter(rows, gidx, dst_ref):
    _sc_kernels()[1](rows, gidx, dst_ref)


def _sc_copy(src):
    return _sc_kernels()[2](src)


def kernel(x, c, w_router, w_ada, b_ada, w_qkv, b_qkv, w_proj, b_proj,
           w_fc1, b_fc1, w_fc2, b_fc2, activate_mod_router):
    tw = _router(x, w_router)[..., 0]
    vals, idx = lax.top_k(tw, K)
    pad = KP - K
    idx_p = jnp.concatenate([idx, jnp.broadcast_to(idx[:, -1:], (B, pad))], axis=1)
    vals_p = jnp.concatenate([vals, jnp.broadcast_to(vals[:, -1:], (B, pad))], axis=1)
    gidx = (idx_p + (jnp.arange(B, dtype=jnp.int32) * N)[:, None]).reshape(TOT)

    x_flat = x.reshape(B * N, C)
    rows = _sc_gather(x_flat, gidx)
    cp = _sc_copy(x_flat)
    topk_x = rows.reshape(B, KP, C)

    qkv, ada = _qkv(topk_x, c.reshape(B, 1, C), w_ada, b_ada, w_qkv, b_qkv)
    o = _attention(qkv)
    out_rows = _mlp(topk_x, o, ada, vals_p.reshape(B, KP, 1),
                    w_proj, b_proj, w_fc1, b_fc1, w_fc2, b_fc2)

    buf = jax.new_ref(cp)
    _sc_scatter(out_rows.reshape(TOT, C), gidx, buf)
    return buf[...].reshape(B, N, C)
